# baseline (speedup 1.0000x reference)
import functools
import jax
import jax.numpy as jnp
from jax import lax
from jax.experimental import pallas as pl
from jax.experimental.pallas import tpu as pltpu
from jax.experimental.pallas import tpu_sc as plsc

N = 10000
NP = 10240
E = 160000
EP = 163840
D = 256
DH = D // 2
NW = 32
EPW_DEG = E // NW
K = 64
CPT = EP // 16 // K
RPT = NP // 16
PAD_DST = 10200

_mesh = plsc.VectorSubcoreMesh(core_axis_name="c", subcore_axis_name="s")


CROWS = EP // K
CPW = CROWS // NW


@functools.partial(
    pl.kernel,
    out_type=jax.ShapeDtypeStruct((2, 2, NP), jnp.float32),
    mesh=_mesh,
    scratch_types=[
        pltpu.VMEM((CPW, K), jnp.int32),
        pltpu.VMEM((CPW, K), jnp.int32),
        pltpu.VMEM((K,), jnp.float32),
        pltpu.VMEM((NP // 16,), jnp.float32),
        pltpu.VMEM_SHARED((NP,), jnp.float32),
        pltpu.VMEM_SHARED((NP,), jnp.float32),
    ],
    compiler_params=pltpu.CompilerParams(needs_layout_passes=False),
)
def _deg_kernel(src_hbm, dst_hbm, out_hbm, src_v, dst_v, onesb, zbuf, hs, hd):
    c = lax.axis_index("c")
    s = lax.axis_index("s")
    wid = s * 2 + c
    base = pl.multiple_of(wid * CPW, 8)
    pltpu.sync_copy(src_hbm.at[pl.ds(base, CPW)], src_v)
    pltpu.sync_copy(dst_hbm.at[pl.ds(base, CPW)], dst_v)

    ones = jnp.ones((16,), jnp.float32)
    z = jnp.zeros((16,), jnp.float32)

    @pl.loop(0, K // 16)
    def _(r):
        onesb[pl.ds(r * 16, 16)] = ones

    @pl.loop(0, NP // 16 // 16)
    def _(r):
        zbuf[pl.ds(r * 16, 16)] = z

    rpt = NP // 16
    off = pl.multiple_of(s * rpt, 8)
    pltpu.sync_copy(zbuf, hs.at[pl.ds(off, rpt)])
    pltpu.sync_copy(zbuf, hd.at[pl.ds(off, rpt)])

    plsc.subcore_barrier()

    @pl.loop(0, CPW)
    def _(ch):
        pltpu.sync_copy(onesb, hs.at[src_v.at[ch]], add=True)
        pltpu.sync_copy(onesb, hd.at[dst_v.at[ch]], add=True)

    plsc.subcore_barrier()

    pltpu.sync_copy(hs.at[pl.ds(off, rpt)], out_hbm.at[c, 0, pl.ds(off, rpt)])
    pltpu.sync_copy(hd.at[pl.ds(off, rpt)], out_hbm.at[c, 1, pl.ds(off, rpt)])


def _mm_body(x_ref, w_ref, degp_ref, ywa_ref, ywb_ref):
    deg = jnp.sum(degp_ref[...], axis=0)
    dsrc = deg[0]
    dis = jnp.where(dsrc > 0, lax.rsqrt(jnp.maximum(dsrc, 1.0)), 0.0)
    xw = jnp.dot(x_ref[...], w_ref[...], preferred_element_type=jnp.float32)
    yw = xw * dis[:, None]
    ywa_ref[...] = yw[:, :DH]
    ywb_ref[...] = yw[:, DH:]


_BM = 512


def _mm_call(x, W, degp):
    grid = (NP // _BM,)
    return pl.pallas_call(
        _mm_body,
        grid=grid,
        in_specs=[
            pl.BlockSpec((_BM, D), lambda i: (i, 0)),
            pl.BlockSpec((D, D), lambda i: (0, 0)),
            pl.BlockSpec((2, 2, _BM), lambda i: (0, 0, i)),
        ],
        out_specs=[pl.BlockSpec((_BM, DH), lambda i: (i, 0)),
                   pl.BlockSpec((_BM, DH), lambda i: (i, 0))],
        out_shape=[jax.ShapeDtypeStruct((N, DH), jnp.float32),
                   jax.ShapeDtypeStruct((N, DH), jnp.float32)],
    )(x, W, degp)


NSEG = 10
SCH = CPT // NSEG
NBUF = 3


@functools.partial(
    pl.kernel,
    out_type=jax.ShapeDtypeStruct((2, NP, DH), jnp.float32),
    mesh=_mesh,
    scratch_types=[
        pltpu.VMEM((SCH, K), jnp.int32),
        pltpu.VMEM((SCH, K), jnp.int32),
        [pltpu.VMEM((K, DH), jnp.float32) for _ in range(NBUF)],
        pltpu.VMEM_SHARED((NP, DH), jnp.float32),
        [pltpu.SemaphoreType.DMA for _ in range(NBUF)],
        [pltpu.SemaphoreType.DMA for _ in range(NBUF)],
    ],
    compiler_params=pltpu.CompilerParams(needs_layout_passes=False),
)
def _edge_kernel(ywa_hbm, ywb_hbm, src_hbm, dst_hbm, out_hbm, src_v, dst_v,
                 bufs, acc, gsems, ssems):
    c = lax.axis_index("c")
    s = lax.axis_index("s")

    z = jnp.zeros((16,), jnp.float32)

    @pl.loop(0, K)
    def _(r):
        for kk in range(DH // 16):
            bufs[0][r, pl.ds(kk * 16, 16)] = z

    @pl.loop(0, RPT // K)
    def _(j):
        pltpu.sync_copy(bufs[0], acc.at[pl.ds(s * RPT + j * K, K)])

    plsc.subcore_barrier()

    def run(yw_hbm):
        def gissue(ch, b):
            pltpu.async_copy(yw_hbm.at[src_v.at[ch]], bufs[b], gsems[b])

        def gwait(b):
            pltpu.make_async_copy(yw_hbm.at[src_v.at[0]], bufs[b],
                                  gsems[b]).wait()

        def sissue(ch, b):
            pltpu.async_copy(bufs[b], acc.at[dst_v.at[ch]], ssems[b],
                             add=True)

        def swait(b):
            pltpu.make_async_copy(bufs[b], acc.at[dst_v.at[0]],
                                  ssems[b]).wait()

        @pl.loop(0, NSEG)
        def _(seg):
            row0 = pl.multiple_of(s * CPT + seg * SCH, 8)
            pltpu.sync_copy(src_hbm.at[pl.ds(row0, SCH)], src_v)
            pltpu.sync_copy(dst_hbm.at[pl.ds(row0, SCH)], dst_v)
            for t in range(SCH + 2):
                if t < SCH:
                    b = t % NBUF
                    if t >= NBUF:
                        swait(b)
                    gissue(t, b)
                d = t - 2
                if d >= 0:
                    bd = d % NBUF
                    gwait(bd)
                    sissue(d, bd)
            for d in range(SCH - NBUF, SCH):
                swait(d % NBUF)

    @pl.when(c == 0)
    def _():
        run(ywa_hbm)

    @pl.when(c == 1)
    def _():
        run(ywb_hbm)

    plsc.subcore_barrier()

    pltpu.sync_copy(acc.at[pl.ds(s * RPT, RPT)],
                    out_hbm.at[c, pl.ds(s * RPT, RPT)])


def _fin_body(acc_ref, degp_ref, b_ref, o_ref):
    deg = jnp.sum(degp_ref[...], axis=0)
    ddst = deg[1]
    dis = jnp.where(ddst > 0, lax.rsqrt(jnp.maximum(ddst, 1.0)), 0.0)
    agg = jnp.concatenate([acc_ref[0], acc_ref[1]], axis=1)
    o_ref[...] = jnp.maximum(agg * dis[:, None] + b_ref[...], 0.0)


def _fin_call(acc, degp, b2):
    grid = (NP // _BM,)
    return pl.pallas_call(
        _fin_body,
        grid=grid,
        in_specs=[
            pl.BlockSpec((2, _BM, DH), lambda i: (0, i, 0)),
            pl.BlockSpec((2, 2, _BM), lambda i: (0, 0, i)),
            pl.BlockSpec((1, D), lambda i: (0, 0)),
        ],
        out_specs=pl.BlockSpec((_BM, D), lambda i: (i, 0)),
        out_shape=jax.ShapeDtypeStruct((N, D), jnp.float32),
    )(acc, degp, b2)


def kernel(x, edge_index, W, b):
    src = edge_index[0]
    dst = edge_index[1]
    pad_sentinel = jnp.full((EP - E,), PAD_DST, jnp.int32)
    srcdeg2d = jnp.concatenate([src, pad_sentinel]).reshape(EP // K, K)
    dst2d = jnp.concatenate([dst, pad_sentinel]).reshape(EP // K, K)
    src2d = jnp.concatenate(
        [src, jnp.zeros((EP - E,), jnp.int32)]).reshape(EP // K, K)
    degp = _deg_kernel(srcdeg2d, dst2d)
    ywa, ywb = _mm_call(x, W, degp)
    acc = _edge_kernel(ywa, ywb, src2d, dst2d)
    out = _fin_call(acc, degp, b.reshape(1, D))
    return out


# speedup vs baseline: 1.0406x; 1.0406x over previous
import functools
import jax
import jax.numpy as jnp
from jax import lax
from jax.experimental import pallas as pl
from jax.experimental.pallas import tpu as pltpu
from jax.experimental.pallas import tpu_sc as plsc

N = 10000
NP = 10240
E = 160000
EP = 163840
D = 256
DH = D // 2
NW = 32
EPW_DEG = E // NW
K = 64
CPT = EP // 16 // K
RPT = NP // 16
PAD_DST = 10200

_mesh = plsc.VectorSubcoreMesh(core_axis_name="c", subcore_axis_name="s")


CROWS = EP // K
CPW = CROWS // NW


@functools.partial(
    pl.kernel,
    out_type=jax.ShapeDtypeStruct((2, 2, NP), jnp.float32),
    mesh=_mesh,
    scratch_types=[
        pltpu.VMEM((CPW, K), jnp.int32),
        pltpu.VMEM((CPW, K), jnp.int32),
        pltpu.VMEM((K,), jnp.float32),
        pltpu.VMEM((NP // 16,), jnp.float32),
        pltpu.VMEM_SHARED((NP,), jnp.float32),
        pltpu.VMEM_SHARED((NP,), jnp.float32),
    ],
    compiler_params=pltpu.CompilerParams(needs_layout_passes=False),
)
def _deg_kernel(src_hbm, dst_hbm, out_hbm, src_v, dst_v, onesb, zbuf, hs, hd):
    c = lax.axis_index("c")
    s = lax.axis_index("s")
    wid = s * 2 + c
    base = pl.multiple_of(wid * CPW, 8)
    pltpu.sync_copy(src_hbm.at[pl.ds(base, CPW)], src_v)
    pltpu.sync_copy(dst_hbm.at[pl.ds(base, CPW)], dst_v)

    ones = jnp.ones((16,), jnp.float32)
    z = jnp.zeros((16,), jnp.float32)

    @pl.loop(0, K // 16)
    def _(r):
        onesb[pl.ds(r * 16, 16)] = ones

    @pl.loop(0, NP // 16 // 16)
    def _(r):
        zbuf[pl.ds(r * 16, 16)] = z

    rpt = NP // 16
    off = pl.multiple_of(s * rpt, 8)
    pltpu.sync_copy(zbuf, hs.at[pl.ds(off, rpt)])
    pltpu.sync_copy(zbuf, hd.at[pl.ds(off, rpt)])

    plsc.subcore_barrier()

    @pl.loop(0, CPW)
    def _(ch):
        pltpu.sync_copy(onesb, hs.at[src_v.at[ch]], add=True)
        pltpu.sync_copy(onesb, hd.at[dst_v.at[ch]], add=True)

    plsc.subcore_barrier()

    pltpu.sync_copy(hs.at[pl.ds(off, rpt)], out_hbm.at[c, 0, pl.ds(off, rpt)])
    pltpu.sync_copy(hd.at[pl.ds(off, rpt)], out_hbm.at[c, 1, pl.ds(off, rpt)])


def _mm_body(x_ref, w_ref, degp_ref, ywa_ref, ywb_ref):
    deg = jnp.sum(degp_ref[...], axis=0)
    dsrc = deg[0]
    dis = jnp.where(dsrc > 0, lax.rsqrt(jnp.maximum(dsrc, 1.0)), 0.0)
    xw = jnp.dot(x_ref[...], w_ref[...], preferred_element_type=jnp.float32)
    yw = xw * dis[:, None]
    ywa_ref[...] = yw[:, :DH]
    ywb_ref[...] = yw[:, DH:]


_BM = 512


def _mm_call(x, W, degp):
    grid = (NP // _BM,)
    return pl.pallas_call(
        _mm_body,
        grid=grid,
        in_specs=[
            pl.BlockSpec((_BM, D), lambda i: (i, 0)),
            pl.BlockSpec((D, D), lambda i: (0, 0)),
            pl.BlockSpec((2, 2, _BM), lambda i: (0, 0, i)),
        ],
        out_specs=[pl.BlockSpec((_BM, DH), lambda i: (i, 0)),
                   pl.BlockSpec((_BM, DH), lambda i: (i, 0))],
        out_shape=[jax.ShapeDtypeStruct((N, DH), jnp.float32),
                   jax.ShapeDtypeStruct((N, DH), jnp.float32)],
    )(x, W, degp)


NSEG = 4
SCH = CPT // NSEG
NBUF = 3


@functools.partial(
    pl.kernel,
    out_type=jax.ShapeDtypeStruct((2, NP, DH), jnp.float32),
    mesh=_mesh,
    scratch_types=[
        pltpu.VMEM((SCH, K), jnp.int32),
        pltpu.VMEM((SCH, K), jnp.int32),
        [pltpu.VMEM((K, DH), jnp.float32) for _ in range(NBUF)],
        pltpu.VMEM_SHARED((NP, DH), jnp.float32),
        [pltpu.SemaphoreType.DMA for _ in range(NBUF)],
        [pltpu.SemaphoreType.DMA for _ in range(NBUF)],
    ],
    compiler_params=pltpu.CompilerParams(needs_layout_passes=False),
)
def _edge_kernel(ywa_hbm, ywb_hbm, src_hbm, dst_hbm, out_hbm, src_v, dst_v,
                 bufs, acc, gsems, ssems):
    c = lax.axis_index("c")
    s = lax.axis_index("s")

    z = jnp.zeros((16,), jnp.float32)

    @pl.loop(0, K)
    def _(r):
        for kk in range(DH // 16):
            bufs[0][r, pl.ds(kk * 16, 16)] = z

    @pl.loop(0, RPT // K)
    def _(j):
        pltpu.sync_copy(bufs[0], acc.at[pl.ds(s * RPT + j * K, K)])

    plsc.subcore_barrier()

    def run(yw_hbm):
        def gissue(ch, b):
            pltpu.async_copy(yw_hbm.at[src_v.at[ch]], bufs[b], gsems[b])

        def gwait(b):
            pltpu.make_async_copy(yw_hbm.at[src_v.at[0]], bufs[b],
                                  gsems[b]).wait()

        def sissue(ch, b):
            pltpu.async_copy(bufs[b], acc.at[dst_v.at[ch]], ssems[b],
                             add=True)

        def swait(b):
            pltpu.make_async_copy(bufs[b], acc.at[dst_v.at[0]],
                                  ssems[b]).wait()

        @pl.loop(0, NSEG)
        def _(seg):
            row0 = pl.multiple_of(s * CPT + seg * SCH, 8)
            pltpu.sync_copy(src_hbm.at[pl.ds(row0, SCH)], src_v)
            pltpu.sync_copy(dst_hbm.at[pl.ds(row0, SCH)], dst_v)
            for t in range(SCH + 2):
                if t < SCH:
                    b = t % NBUF
                    if t >= NBUF:
                        swait(b)
                    gissue(t, b)
                d = t - 2
                if d >= 0:
                    bd = d % NBUF
                    gwait(bd)
                    sissue(d, bd)
            for d in range(SCH - NBUF, SCH):
                swait(d % NBUF)

    @pl.when(c == 0)
    def _():
        run(ywa_hbm)

    @pl.when(c == 1)
    def _():
        run(ywb_hbm)

    plsc.subcore_barrier()

    pltpu.sync_copy(acc.at[pl.ds(s * RPT, RPT)],
                    out_hbm.at[c, pl.ds(s * RPT, RPT)])


def _fin_body(acc_ref, degp_ref, b_ref, o_ref):
    deg = jnp.sum(degp_ref[...], axis=0)
    ddst = deg[1]
    dis = jnp.where(ddst > 0, lax.rsqrt(jnp.maximum(ddst, 1.0)), 0.0)
    agg = jnp.concatenate([acc_ref[0], acc_ref[1]], axis=1)
    o_ref[...] = jnp.maximum(agg * dis[:, None] + b_ref[...], 0.0)


def _fin_call(acc, degp, b2):
    grid = (NP // _BM,)
    return pl.pallas_call(
        _fin_body,
        grid=grid,
        in_specs=[
            pl.BlockSpec((2, _BM, DH), lambda i: (0, i, 0)),
            pl.BlockSpec((2, 2, _BM), lambda i: (0, 0, i)),
            pl.BlockSpec((1, D), lambda i: (0, 0)),
        ],
        out_specs=pl.BlockSpec((_BM, D), lambda i: (i, 0)),
        out_shape=jax.ShapeDtypeStruct((N, D), jnp.float32),
    )(acc, degp, b2)


def kernel(x, edge_index, W, b):
    src = edge_index[0]
    dst = edge_index[1]
    pad_sentinel = jnp.full((EP - E,), PAD_DST, jnp.int32)
    srcdeg2d = jnp.concatenate([src, pad_sentinel]).reshape(EP // K, K)
    dst2d = jnp.concatenate([dst, pad_sentinel]).reshape(EP // K, K)
    src2d = jnp.concatenate(
        [src, jnp.zeros((EP - E,), jnp.int32)]).reshape(EP // K, K)
    degp = _deg_kernel(srcdeg2d, dst2d)
    ywa, ywb = _mm_call(x, W, degp)
    acc = _edge_kernel(ywa, ywb, src2d, dst2d)
    out = _fin_call(acc, degp, b.reshape(1, D))
    return out
